# baseline (speedup 1.0000x reference)
import functools

import jax
import jax.numpy as jnp
from jax.experimental import pallas as pl
from jax.experimental.pallas import tpu as pltpu

_INV_TEMP = 10.0
_EPS_SQ = 1e-24


def _smog_logits_kernel(x_ref, g_ref, out_ref):
    x = x_ref[...]
    g = g_ref[...]
    xs = x * jax.lax.rsqrt(
        jnp.maximum(jnp.sum(x * x, axis=1, keepdims=True), _EPS_SQ))
    gs = g * jax.lax.rsqrt(
        jnp.maximum(jnp.sum(g * g, axis=1, keepdims=True), _EPS_SQ))
    acc = jax.lax.dot_general(
        xs, gs, (((1,), (1,)), ((), ())),
        preferred_element_type=jnp.float32)
    out_ref[...] = acc * _INV_TEMP


@functools.partial(jax.jit, static_argnames=("bm", "bn"))
def _smog_logits(x, group_features, bm, bn):
    b, d = x.shape
    k, _ = group_features.shape
    bm = min(bm, b)
    bn = min(bn, k)
    return pl.pallas_call(
        _smog_logits_kernel,
        grid=(b // bm, k // bn),
        in_specs=[
            pl.BlockSpec((bm, d), lambda i, j: (i, 0)),
            pl.BlockSpec((bn, d), lambda i, j: (j, 0)),
        ],
        out_specs=pl.BlockSpec((bm, bn), lambda i, j: (i, j)),
        out_shape=jax.ShapeDtypeStruct((b, k), jnp.float32),
        compiler_params=pltpu.CompilerParams(
            dimension_semantics=("parallel", "parallel")),
    )(x, group_features)


def kernel(x, group_features):
    return _smog_logits(x, group_features, bm=1024, bn=2048)


# speedup vs baseline: 1.1552x; 1.1552x over previous
import functools

import jax
import jax.numpy as jnp
from jax.experimental import pallas as pl
from jax.experimental.pallas import tpu as pltpu

_INV_TEMP = 10.0
_EPS_SQ = 1e-24


def _smog_logits_kernel(x_ref, g_ref, out_ref):
    x = x_ref[...]
    g = g_ref[...]
    xs = x * jax.lax.rsqrt(
        jnp.maximum(jnp.sum(x * x, axis=1, keepdims=True), _EPS_SQ))
    gs = g * jax.lax.rsqrt(
        jnp.maximum(jnp.sum(g * g, axis=1, keepdims=True), _EPS_SQ))
    acc = jax.lax.dot_general(
        xs, gs, (((1,), (1,)), ((), ())),
        preferred_element_type=jnp.float32)
    out_ref[...] = acc * _INV_TEMP


@functools.partial(jax.jit, static_argnames=("bm", "bn"))
def _smog_logits(x, group_features, bm, bn):
    b, d = x.shape
    k, _ = group_features.shape
    bm = min(bm, b)
    bn = min(bn, k)
    return pl.pallas_call(
        _smog_logits_kernel,
        grid=(b // bm, k // bn),
        in_specs=[
            pl.BlockSpec((bm, d), lambda i, j: (i, 0)),
            pl.BlockSpec((bn, d), lambda i, j: (j, 0)),
        ],
        out_specs=pl.BlockSpec((bm, bn), lambda i, j: (i, j)),
        out_shape=jax.ShapeDtypeStruct((b, k), jnp.float32),
        compiler_params=pltpu.CompilerParams(
            dimension_semantics=("parallel", "parallel")),
    )(x, group_features)


def kernel(x, group_features):
    return _smog_logits(x, group_features, bm=512, bn=8192)


# speedup vs baseline: 1.1580x; 1.0025x over previous
import functools

import jax
import jax.numpy as jnp
from jax.experimental import pallas as pl
from jax.experimental.pallas import tpu as pltpu

_INV_TEMP = 10.0
_EPS_SQ = 1e-24


def _smog_logits_kernel(x_ref, g_ref, out_ref):
    x = x_ref[...]
    g = g_ref[...]
    xs = x * jax.lax.rsqrt(
        jnp.maximum(jnp.sum(x * x, axis=1, keepdims=True), _EPS_SQ))
    gs = g * jax.lax.rsqrt(
        jnp.maximum(jnp.sum(g * g, axis=1, keepdims=True), _EPS_SQ))
    acc = jax.lax.dot_general(
        xs, gs, (((1,), (1,)), ((), ())),
        preferred_element_type=jnp.float32)
    out_ref[...] = acc * _INV_TEMP


@functools.partial(jax.jit, static_argnames=("bm", "bn"))
def _smog_logits(x, group_features, bm, bn):
    b, d = x.shape
    k, _ = group_features.shape
    bm = min(bm, b)
    bn = min(bn, k)
    return pl.pallas_call(
        _smog_logits_kernel,
        grid=(b // bm, k // bn),
        in_specs=[
            pl.BlockSpec((bm, d), lambda i, j: (i, 0)),
            pl.BlockSpec((bn, d), lambda i, j: (j, 0)),
        ],
        out_specs=pl.BlockSpec((bm, bn), lambda i, j: (i, j)),
        out_shape=jax.ShapeDtypeStruct((b, k), jnp.float32),
        compiler_params=pltpu.CompilerParams(
            dimension_semantics=("parallel", "parallel")),
    )(x, group_features)


def kernel(x, group_features):
    return _smog_logits(x, group_features, bm=256, bn=8192)
